# baseline (speedup 1.0000x reference)
import functools

import jax
import jax.numpy as jnp
from jax import lax
from jax.experimental import pallas as pl
from jax.experimental.pallas import tpu as pltpu
from jax.experimental.pallas import tpu_sc as plsc

BATCH = 4096
N_TOKENS = 200
D_MODEL = 64
NC, NS, L = 2, 16, 16
NW = NC * NS
ROWS_PER_W = BATCH // NW
HALF = N_TOKENS // 2


def _body(tokens_hbm, table_hbm, pos_hbm, out_hbm, idx_v, rows_v, pos_v, sem):
    wid = lax.axis_index("s") * NC + lax.axis_index("c")
    row0 = wid * ROWS_PER_W

    pltpu.sync_copy(pos_hbm, pos_v)

    def one_row(c, _):
        r = row0 + c
        pltpu.sync_copy(tokens_hbm.at[r], idx_v)
        g0 = pltpu.async_copy(table_hbm.at[idx_v.at[0]],
                              rows_v.at[pl.ds(0, HALF)], sem)
        g1 = pltpu.async_copy(table_hbm.at[idx_v.at[1]],
                              rows_v.at[pl.ds(HALF, HALF)], sem)
        g0.wait()
        g1.wait()

        def add_pos(i, _):
            for j in range(D_MODEL // L):
                sl = pl.ds(j * L, L)
                plsc.addupdate(rows_v.at[i, sl], pos_v[i, sl])
            return 0

        lax.fori_loop(0, N_TOKENS, add_pos, 0)
        pltpu.sync_copy(rows_v, out_hbm.at[r])
        return 0

    lax.fori_loop(0, ROWS_PER_W, one_row, 0)


def kernel(tokens, token_embedding, position_embedding):
    tokens3 = tokens.reshape(BATCH, 2, HALF)
    mesh = plsc.VectorSubcoreMesh(core_axis_name="c", subcore_axis_name="s",
                                  num_cores=NC, num_subcores=NS)
    run = pl.kernel(
        _body,
        out_type=jax.ShapeDtypeStruct((BATCH, N_TOKENS, D_MODEL), jnp.float32),
        mesh=mesh,
        compiler_params=pltpu.CompilerParams(use_tc_tiling_on_sc=False),
        scratch_types=[
            pltpu.VMEM((2, HALF), jnp.int32),
            pltpu.VMEM((N_TOKENS, D_MODEL), jnp.float32),
            pltpu.VMEM((N_TOKENS, D_MODEL), jnp.float32),
            pltpu.SemaphoreType.DMA,
        ],
    )
    return run(tokens3, token_embedding, position_embedding)


# speedup vs baseline: 1.1870x; 1.1870x over previous
import jax
import jax.numpy as jnp
from jax import lax
from jax.experimental import pallas as pl
from jax.experimental.pallas import tpu as pltpu
from jax.experimental.pallas import tpu_sc as plsc

BATCH = 4096
N_TOKENS = 200
D_MODEL = 64
NC, NS, L = 2, 16, 16
NW = NC * NS
ROWS_PER_W = BATCH // NW
HALF = N_TOKENS // 2
K = 4
NGROUPS = ROWS_PER_W // K


def _body(tokens_hbm, table_hbm, pos_hbm, out_hbm,
          idx_v, rows_v, pos_v, sem_g0, sem_g1, sem_w0, sem_w1):
    wid = lax.axis_index("s") * NC + lax.axis_index("c")
    row0 = wid * ROWS_PER_W
    sems_g = (sem_g0, sem_g1)
    sems_w = (sem_w0, sem_w1)

    def load_idx(bank, g):
        pltpu.sync_copy(tokens_hbm.at[pl.ds(row0 + g * K, K)], idx_v.at[bank])

    def fire_gathers(bank, g):
        del g
        for b in range(K):
            pltpu.async_copy(table_hbm.at[idx_v.at[bank, b, 0]],
                             rows_v.at[bank, b, pl.ds(0, HALF)], sems_g[bank])
            pltpu.async_copy(table_hbm.at[idx_v.at[bank, b, 1]],
                             rows_v.at[bank, b, pl.ds(HALF, HALF)],
                             sems_g[bank])

    def drain_gathers(bank):
        for b in range(K):
            pltpu.make_async_copy(table_hbm.at[idx_v.at[bank, b, 0]],
                                  rows_v.at[bank, b, pl.ds(0, HALF)],
                                  sems_g[bank]).wait()
            pltpu.make_async_copy(table_hbm.at[idx_v.at[bank, b, 1]],
                                  rows_v.at[bank, b, pl.ds(HALF, HALF)],
                                  sems_g[bank]).wait()

    def add_and_writeback(bank, g):
        for b in range(K):
            def add_pos(k, _):
                for jj in range(4):
                    i = k * 4 + jj
                    for j in range(D_MODEL // L):
                        sl = pl.ds(j * L, L)
                        plsc.addupdate(rows_v.at[bank, b, i, sl],
                                       pos_v[i, sl])
                return 0

            lax.fori_loop(0, N_TOKENS // 4, add_pos, 0)
            pltpu.async_copy(rows_v.at[bank, b],
                             out_hbm.at[row0 + g * K + b], sems_w[bank])

    def drain_writebacks(bank):
        for b in range(K):
            pltpu.make_async_copy(rows_v.at[bank, b], out_hbm.at[row0],
                                  sems_w[bank]).wait()

    pltpu.sync_copy(pos_hbm, pos_v)

    load_idx(0, 0)
    fire_gathers(0, 0)
    load_idx(1, 1)
    fire_gathers(1, 1)
    drain_gathers(0)
    add_and_writeback(0, 0)

    def pair(gp, _):
        g = 2 * gp + 1
        drain_writebacks(0)
        load_idx(0, g + 1)
        fire_gathers(0, g + 1)
        drain_gathers(1)
        add_and_writeback(1, g)

        g2 = g + 1
        drain_writebacks(1)
        load_idx(1, g2 + 1)
        fire_gathers(1, g2 + 1)
        drain_gathers(0)
        add_and_writeback(0, g2)
        return 0

    lax.fori_loop(0, (NGROUPS - 2) // 2, pair, 0)

    drain_writebacks(0)
    drain_gathers(1)
    add_and_writeback(1, NGROUPS - 1)
    drain_writebacks(1)


def kernel(tokens, token_embedding, position_embedding):
    tokens3 = tokens.reshape(BATCH, 2, HALF)
    mesh = plsc.VectorSubcoreMesh(core_axis_name="c", subcore_axis_name="s",
                                  num_cores=NC, num_subcores=NS)
    run = pl.kernel(
        _body,
        out_type=jax.ShapeDtypeStruct((BATCH, N_TOKENS, D_MODEL), jnp.float32),
        mesh=mesh,
        compiler_params=pltpu.CompilerParams(use_tc_tiling_on_sc=False),
        scratch_types=[
            pltpu.VMEM((2, K, 2, HALF), jnp.int32),
            pltpu.VMEM((2, K, N_TOKENS, D_MODEL), jnp.float32),
            pltpu.VMEM((N_TOKENS, D_MODEL), jnp.float32),
            pltpu.SemaphoreType.DMA,
            pltpu.SemaphoreType.DMA,
            pltpu.SemaphoreType.DMA,
            pltpu.SemaphoreType.DMA,
        ],
    )
    return run(tokens3, token_embedding, position_embedding)
